# baseline (speedup 1.0000x reference)
import functools

import jax
import jax.numpy as jnp
from jax import lax
from jax.experimental import pallas as pl
from jax.experimental.pallas import tpu as pltpu
from jax.experimental.pallas import tpu_sc as plsc

NC, NS, L = 2, 16, 16
NW = NC * NS
HIDDEN = 128
UNROLL = 4


def _rsqrt(x):
    xi = plsc.bitcast(x, jnp.int32)
    yi = jnp.int32(0x5F3759DF) - (xi >> 1)
    y = plsc.bitcast(yi, jnp.float32)
    for _ in range(3):
        y = y * (1.5 - 0.5 * x * y * y)
    return y


def _make_sc_kernel(n_tokens, seq_len):
    b_per_w = n_tokens // NW
    n_groups = b_per_w // L
    mesh = plsc.VectorSubcoreMesh(
        core_axis_name="c", subcore_axis_name="s", num_cores=NC, num_subcores=NS
    )

    @functools.partial(
        pl.kernel,
        mesh=mesh,
        compiler_params=pltpu.CompilerParams(needs_layout_passes=False),
        out_type=jax.ShapeDtypeStruct((n_tokens, HIDDEN), jnp.float32),
        scratch_types=[
            pltpu.VMEM((b_per_w,), jnp.int32),
            pltpu.VMEM((b_per_w,), jnp.int32),
            pltpu.VMEM((b_per_w, HIDDEN), jnp.float32),
            pltpu.VMEM((b_per_w, HIDDEN), jnp.float32),
            pltpu.VMEM((b_per_w, HIDDEN), jnp.float32),
            pltpu.VMEM((HIDDEN,), jnp.float32),
            pltpu.VMEM((HIDDEN,), jnp.float32),
            pltpu.SemaphoreType.DMA,
            pltpu.SemaphoreType.DMA,
        ],
    )
    def sc_kernel(ids_hbm, tt_hbm, token_hbm, pos_hbm, type_hbm, g_hbm, b_hbm,
                  out_hbm, idx_v, tt_v, rows_v, pos_v, typ_v, g_v, b_v,
                  sem0, sem1):
        wid = lax.axis_index("s") * NC + lax.axis_index("c")
        base = wid * b_per_w

        pltpu.sync_copy(ids_hbm.at[pl.ds(base, b_per_w)], idx_v)
        pltpu.sync_copy(tt_hbm.at[pl.ds(base, b_per_w)], tt_v)

        copies = []
        for j in range(b_per_w // 128):
            sem = sem0 if j % 2 == 0 else sem1
            copies.append(pltpu.async_copy(
                token_hbm.at[idx_v.at[pl.ds(j * 128, 128)]],
                rows_v.at[pl.ds(j * 128, 128)], sem))
            copies.append(pltpu.async_copy(
                type_hbm.at[tt_v.at[pl.ds(j * 128, 128)]],
                typ_v.at[pl.ds(j * 128, 128)], sem))

        pos_base = lax.rem(base, seq_len)
        pltpu.sync_copy(pos_hbm.at[pl.ds(pos_base, b_per_w)], pos_v)
        pltpu.sync_copy(g_hbm, g_v)
        pltpu.sync_copy(b_hbm, b_v)
        for c in copies:
            c.wait()

        inv_h = jnp.float32(1.0 / HIDDEN)
        zeros = jnp.zeros((L,), jnp.float32)

        def group_body(g, _):
            tok = g * L + lax.iota(jnp.int32, L)

            def acc_body(j, carry):
                s0, s1, q0, q1 = carry
                for u in range(UNROLL):
                    h = j * UNROLL + u
                    hs = jnp.full((L,), h, jnp.int32)
                    v = (plsc.load_gather(rows_v, [tok, hs])
                         + plsc.load_gather(pos_v, [tok, hs])
                         + plsc.load_gather(typ_v, [tok, hs]))
                    plsc.store_scatter(rows_v, [tok, hs], v)
                    if u % 2 == 0:
                        s0 = s0 + v
                        q0 = q0 + v * v
                    else:
                        s1 = s1 + v
                        q1 = q1 + v * v
                return s0, s1, q0, q1

            s0, s1, q0, q1 = lax.fori_loop(
                0, HIDDEN // UNROLL, acc_body, (zeros, zeros, zeros, zeros))
            mean = (s0 + s1) * inv_h
            var = (q0 + q1) * inv_h - mean * mean
            inv = _rsqrt(var + 1e-12)

            def norm_body(j, _):
                for u in range(UNROLL):
                    h = j * UNROLL + u
                    hs = jnp.full((L,), h, jnp.int32)
                    v = plsc.load_gather(rows_v, [tok, hs])
                    gh = plsc.load_gather(g_v, [hs])
                    bh = plsc.load_gather(b_v, [hs])
                    out = (v - mean) * inv * gh + bh
                    plsc.store_scatter(rows_v, [tok, hs], out)
                return 0

            lax.fori_loop(0, HIDDEN // UNROLL, norm_body, 0)
            return 0

        lax.fori_loop(0, n_groups, group_body, 0)

        pltpu.sync_copy(rows_v, out_hbm.at[pl.ds(base, b_per_w)])

    return sc_kernel


def kernel(input_ids, token_type_ids, token_table, pos_table, type_table,
           ln_gamma, ln_beta):
    b, s = input_ids.shape
    n = b * s
    ids = input_ids.reshape(n).astype(jnp.int32)
    tt = token_type_ids.reshape(n).astype(jnp.int32)
    sc = _make_sc_kernel(n, s)
    out = sc(ids, tt, token_table, pos_table, type_table, ln_gamma, ln_beta)
    return out.reshape(b, s, HIDDEN)


# speedup vs baseline: 9.6729x; 9.6729x over previous
import functools

import jax
import jax.numpy as jnp
from jax import lax
from jax.experimental import pallas as pl
from jax.experimental.pallas import tpu as pltpu
from jax.experimental.pallas import tpu_sc as plsc

NC, NS, L = 2, 16, 16
NW = NC * NS
HIDDEN = 128
ROWS_PER_STEP = 2048


def _make_sc_gather(n_tokens):
    b_per_w = n_tokens // NW
    mesh = plsc.VectorSubcoreMesh(
        core_axis_name="c", subcore_axis_name="s", num_cores=NC, num_subcores=NS
    )

    @functools.partial(
        pl.kernel,
        mesh=mesh,
        compiler_params=pltpu.CompilerParams(needs_layout_passes=False),
        out_type=jax.ShapeDtypeStruct((n_tokens, HIDDEN), jnp.float32),
        scratch_types=[
            pltpu.VMEM((b_per_w,), jnp.int32),
            pltpu.VMEM((b_per_w, HIDDEN), jnp.float32),
            pltpu.SemaphoreType.DMA,
            pltpu.SemaphoreType.DMA,
        ],
    )
    def sc_gather(ids_hbm, token_hbm, out_hbm, idx_v, rows_v, sem0, sem1):
        wid = lax.axis_index("s") * NC + lax.axis_index("c")
        base = wid * b_per_w
        pltpu.sync_copy(ids_hbm.at[pl.ds(base, b_per_w)], idx_v)
        copies = []
        for j in range(b_per_w // 128):
            sem = sem0 if j % 2 == 0 else sem1
            copies.append(pltpu.async_copy(
                token_hbm.at[idx_v.at[pl.ds(j * 128, 128)]],
                rows_v.at[pl.ds(j * 128, 128)], sem))
        for c in copies:
            c.wait()
        pltpu.sync_copy(rows_v, out_hbm.at[pl.ds(base, b_per_w)])

    return sc_gather


def _tc_ln_body(x_ref, pos_ref, ttf_ref, type_ref, g_ref, b_ref, o_ref):
    x = x_ref[...]
    t0 = type_ref[0:1, :]
    t1 = type_ref[1:2, :]
    e = x + pos_ref[...] + t0 + ttf_ref[...] * (t1 - t0)
    mean = jnp.mean(e, axis=-1, keepdims=True)
    c = e - mean
    var = jnp.mean(c * c, axis=-1, keepdims=True)
    o_ref[...] = c * lax.rsqrt(var + 1e-12) * g_ref[...] + b_ref[...]


def _tc_ln(gathered, pos_table, ttf, type_table, gamma, beta, seq_len):
    n = gathered.shape[0]
    r = ROWS_PER_STEP
    grid = n // r
    pos_blocks = seq_len // r if seq_len >= r else 1
    return pl.pallas_call(
        _tc_ln_body,
        grid=(grid,),
        in_specs=[
            pl.BlockSpec((r, HIDDEN), lambda g: (g, 0)),
            pl.BlockSpec((min(r, seq_len), HIDDEN), lambda g: (g % pos_blocks, 0)),
            pl.BlockSpec((r, 1), lambda g: (g, 0)),
            pl.BlockSpec((2, HIDDEN), lambda g: (0, 0)),
            pl.BlockSpec((1, HIDDEN), lambda g: (0, 0)),
            pl.BlockSpec((1, HIDDEN), lambda g: (0, 0)),
        ],
        out_specs=pl.BlockSpec((r, HIDDEN), lambda g: (g, 0)),
        out_shape=jax.ShapeDtypeStruct((n, HIDDEN), jnp.float32),
    )(gathered, pos_table, ttf, type_table, gamma, beta)


def kernel(input_ids, token_type_ids, token_table, pos_table, type_table,
           ln_gamma, ln_beta):
    b, s = input_ids.shape
    n = b * s
    ids = input_ids.reshape(n).astype(jnp.int32)
    ttf = token_type_ids.reshape(n, 1).astype(jnp.float32)
    gathered = _make_sc_gather(n)(ids, token_table)
    out = _tc_ln(gathered, pos_table, ttf, type_table,
                 ln_gamma.reshape(1, HIDDEN), ln_beta.reshape(1, HIDDEN), s)
    return out.reshape(b, s, HIDDEN)
